# Initial kernel scaffold; baseline (speedup 1.0000x reference)
import functools

import jax
import jax.numpy as jnp
from jax import lax
from jax.experimental import pallas as pl
from jax.experimental.pallas import tpu as pltpu
from jax.experimental.pallas import tpu_sc as plsc

N = 10000
E = 320000
D = 128
NC = 2
NS = 16
NW = NC * NS
EPW = E // NW
CHUNK = 125
NCHUNK = EPW // CHUNK
HROWS = 80
STRIPE = N // NS

_mesh = plsc.VectorSubcoreMesh(core_axis_name="c", subcore_axis_name="s")


def _hist_body(dst_hbm, out_hbm, dvals, hist, idrow, shist):
    cid = lax.axis_index("c")
    sid = lax.axis_index("s")
    tid = cid * NS + sid

    zeros16 = jnp.zeros((16,), jnp.float32)

    def zb(i, carry):
        hist[i >> 3, pl.ds((i & 7) * 16, 16)] = zeros16
        return carry
    lax.fori_loop(0, HROWS * 8, zb, 0)

    def ib(j, carry):
        idrow[pl.ds(j * 16, 16)] = lax.iota(jnp.int32, 16) + j * 16
        return carry
    lax.fori_loop(0, HROWS // 16, ib, 0)

    @pl.when(sid == 0)
    def _():
        pltpu.sync_copy(hist, shist)

    pltpu.sync_copy(dst_hbm.at[tid], dvals)
    plsc.subcore_barrier()

    ones16 = jnp.ones((16,), jnp.float32)

    def hb(i, carry):
        d = dvals[pl.ds(i * 16, 16)]
        plsc.addupdate_scatter(hist, [d >> 7, d & 127], ones16)
        return carry
    lax.fori_loop(0, EPW // 16, hb, 0)

    pltpu.sync_copy(hist, shist.at[idrow], add=True)
    plsc.subcore_barrier()

    rows_per_tile = HROWS // NS
    pltpu.sync_copy(
        shist.at[pl.ds(sid * rows_per_tile, rows_per_tile)],
        out_hbm.at[cid, pl.ds(sid * rows_per_tile, rows_per_tile)],
    )


_hist_call = pl.kernel(
    _hist_body,
    out_type=jax.ShapeDtypeStruct((NC, HROWS, 128), jnp.float32),
    mesh=_mesh,
    scratch_types=[
        pltpu.VMEM((EPW,), jnp.int32),
        pltpu.VMEM((HROWS, 128), jnp.float32),
        pltpu.VMEM((HROWS,), jnp.int32),
        pltpu.VMEM_SHARED((HROWS, 128), jnp.float32),
    ],
)


def _scatter_body(xt_hbm, src_hbm, dst_hbm, out_hbm,
                  src_v, dst_v, rows_v, zbuf, sem, h_sh):
    cid = lax.axis_index("c")
    sid = lax.axis_index("s")
    tid = cid * NS + sid

    pltpu.sync_copy(src_hbm.at[tid], src_v)
    pltpu.sync_copy(dst_hbm.at[tid], dst_v)

    @pl.when(cid == 0)
    def _():
        pltpu.sync_copy(
            xt_hbm.at[pl.ds(sid * STRIPE, STRIPE)],
            h_sh.at[pl.ds(sid * STRIPE, STRIPE)],
        )

    @pl.when(cid == 1)
    def _():
        zeros16 = jnp.zeros((16,), jnp.float32)

        def zb(i, carry):
            zbuf[i >> 3, pl.ds((i & 7) * 16, 16)] = zeros16
            return carry
        lax.fori_loop(0, CHUNK * 8, zb, 0)
        for k in range(STRIPE // CHUNK):
            pltpu.sync_copy(zbuf, h_sh.at[pl.ds(sid * STRIPE + k * CHUNK, CHUNK)])

    plsc.subcore_barrier()

    def cb(c, carry):
        pltpu.async_copy(xt_hbm.at[src_v.at[c]], rows_v, sem).wait()
        pltpu.sync_copy(rows_v, h_sh.at[dst_v.at[c]], add=True)
        return carry
    lax.fori_loop(0, NCHUNK, cb, 0)

    plsc.subcore_barrier()
    pltpu.sync_copy(
        h_sh.at[pl.ds(sid * STRIPE, STRIPE)],
        out_hbm.at[cid, pl.ds(sid * STRIPE, STRIPE)],
    )


_scatter_call = pl.kernel(
    _scatter_body,
    out_type=jax.ShapeDtypeStruct((NC, N, D), jnp.float32),
    mesh=_mesh,
    scratch_types=[
        pltpu.VMEM((NCHUNK, CHUNK), jnp.int32),
        pltpu.VMEM((NCHUNK, CHUNK), jnp.int32),
        pltpu.VMEM((CHUNK, D), jnp.float32),
        pltpu.VMEM((CHUNK, D), jnp.float32),
        pltpu.SemaphoreType.DMA,
        pltpu.VMEM_SHARED((N, D), jnp.float32),
    ],
)


def _norm_body(x_ref, p0_ref, p1_ref, xt_ref, dis_ref):
    deg = p0_ref[...] + p1_ref[...] + 1.0
    dis = lax.rsqrt(deg)
    dis_ref[...] = dis
    xt_ref[...] = x_ref[...] * dis


def _out_body(x_ref, p0_ref, p1_ref, dis_ref, wt_ref, b_ref, o_ref):
    s = (p0_ref[...] + p1_ref[...]) * dis_ref[...]
    h = jnp.dot(s, wt_ref[...], preferred_element_type=jnp.float32) + b_ref[...]
    o_ref[...] = x_ref[...] + jnp.maximum(h, 0.0)


_RB = 1000


def kernel(x, edge_index, W, b):
    src = edge_index[0].astype(jnp.int32)
    dst = edge_index[1].astype(jnp.int32)
    src3 = src.reshape(NW, NCHUNK, CHUNK)
    dst3 = dst.reshape(NW, NCHUNK, CHUNK)
    dst2 = dst.reshape(NW, EPW)

    hp = _hist_call(dst2)
    p = hp.reshape(NC, HROWS * 128)[:, :N].reshape(NC, N, 1)

    grid = N // _RB
    row_spec = pl.BlockSpec((_RB, D), lambda i: (i, 0))
    col_spec = pl.BlockSpec((_RB, 1), lambda i: (i, 0))

    xt, dis = pl.pallas_call(
        _norm_body,
        grid=(grid,),
        in_specs=[row_spec, col_spec, col_spec],
        out_specs=[row_spec, col_spec],
        out_shape=[
            jax.ShapeDtypeStruct((N, D), jnp.float32),
            jax.ShapeDtypeStruct((N, 1), jnp.float32),
        ],
    )(x, p[0], p[1])

    parts = _scatter_call(xt, src3, dst3)

    out = pl.pallas_call(
        _out_body,
        grid=(grid,),
        in_specs=[
            row_spec, row_spec, row_spec, col_spec,
            pl.BlockSpec((D, D), lambda i: (0, 0)),
            pl.BlockSpec((1, D), lambda i: (0, 0)),
        ],
        out_specs=row_spec,
        out_shape=jax.ShapeDtypeStruct((N, D), jnp.float32),
    )(x, parts[0], parts[1], dis, W.T, b.reshape(1, D))
    return out


# speedup vs baseline: 18.9486x; 18.9486x over previous
import functools

import jax
import jax.numpy as jnp
from jax import lax
from jax.experimental import pallas as pl
from jax.experimental.pallas import tpu as pltpu
from jax.experimental.pallas import tpu_sc as plsc

N = 10000
E = 320000
D = 128
NC = 2
NS = 16
NW = NC * NS
EPW = E // NW
CHUNK = 125
NCHUNK = EPW // CHUNK
HROWS = 80
NPAD = 10240
TSTRIPE = NPAD // NS
LASTROWS = N - 15 * TSTRIPE
ZROWS = 128

def _hist_body(dst_hbm, out_hbm, dvals, hist):
    cid = lax.axis_index("c")
    sid = lax.axis_index("s")
    tid = cid * NS + sid

    zeros16 = jnp.zeros((16,), jnp.float32)

    def zb(i, carry):
        hist[pl.ds(i * 16, 16)] = zeros16
        return carry
    lax.fori_loop(0, NPAD // 16, zb, 0)

    pltpu.sync_copy(dst_hbm.at[tid], dvals)

    ones16 = jnp.ones((16,), jnp.float32)

    def hb(i, carry):
        d = dvals[pl.ds(i * 16, 16)]
        plsc.addupdate_scatter(hist, [d], ones16)
        return carry
    lax.fori_loop(0, EPW // 16, hb, 0)

    pltpu.sync_copy(hist, out_hbm.at[tid])


@functools.cache
def _sc_calls():
    mesh = plsc.VectorSubcoreMesh(
        core_axis_name="c", subcore_axis_name="s", num_cores=NC, num_subcores=NS
    )
    params = pltpu.CompilerParams(needs_layout_passes=False)
    hist_call = pl.kernel(
        _hist_body,
        out_type=jax.ShapeDtypeStruct((NW, NPAD), jnp.float32),
        mesh=mesh,
        compiler_params=params,
        scratch_types=[
            pltpu.VMEM((EPW,), jnp.int32),
            pltpu.VMEM((NPAD,), jnp.float32),
        ],
    )
    scatter_call = pl.kernel(
        _scatter_body,
        out_type=jax.ShapeDtypeStruct((NC, N, D), jnp.float32),
        mesh=mesh,
        compiler_params=params,
        scratch_types=[
            pltpu.VMEM((NCHUNK, CHUNK), jnp.int32),
            pltpu.VMEM((NCHUNK, CHUNK), jnp.int32),
            pltpu.VMEM((CHUNK, D), jnp.float32),
            pltpu.SemaphoreType.DMA,
            pltpu.VMEM_SHARED((NPAD, D), jnp.float32),
        ],
    )
    return hist_call, scatter_call


def _scatter_body(xt_hbm, src_hbm, dst_hbm, out_hbm,
                  src_v, dst_v, rows_v, sem, h_sh):
    cid = lax.axis_index("c")
    sid = lax.axis_index("s")
    tid = cid * NS + sid

    pltpu.sync_copy(src_hbm.at[tid], src_v)
    pltpu.sync_copy(dst_hbm.at[tid], dst_v)

    @pl.when(sid < NS - 1)
    def _():
        pltpu.sync_copy(
            xt_hbm.at[pl.ds(sid * TSTRIPE, TSTRIPE)],
            h_sh.at[pl.ds(sid * TSTRIPE, TSTRIPE)],
        )

    @pl.when(sid == NS - 1)
    def _():
        pltpu.sync_copy(
            xt_hbm.at[pl.ds((NS - 1) * TSTRIPE, LASTROWS)],
            h_sh.at[pl.ds((NS - 1) * TSTRIPE, LASTROWS)],
        )

    plsc.subcore_barrier()

    def cb(c, carry):
        pltpu.async_copy(xt_hbm.at[src_v.at[c]], rows_v, sem).wait()
        pltpu.sync_copy(rows_v, h_sh.at[dst_v.at[c]], add=True)
        return carry
    lax.fori_loop(0, NCHUNK, cb, 0)

    plsc.subcore_barrier()

    @pl.when(sid < NS - 1)
    def _():
        pltpu.sync_copy(
            h_sh.at[pl.ds(sid * TSTRIPE, TSTRIPE)],
            out_hbm.at[cid, pl.ds(sid * TSTRIPE, TSTRIPE)],
        )

    @pl.when(sid == NS - 1)
    def _():
        pltpu.sync_copy(
            h_sh.at[pl.ds((NS - 1) * TSTRIPE, LASTROWS)],
            out_hbm.at[cid, pl.ds((NS - 1) * TSTRIPE, LASTROWS)],
        )


def _norm_body(x_ref, p_ref, xt_ref, dis_ref):
    deg = jnp.sum(p_ref[...], axis=0) + 1.0
    dis = lax.rsqrt(deg)
    dis_ref[...] = dis
    xt_ref[...] = x_ref[...] * dis


def _out_body(x_ref, p0_ref, p1_ref, xt_ref, dis_ref, wt_ref, b_ref, o_ref):
    s = (p0_ref[...] + p1_ref[...] - xt_ref[...]) * dis_ref[...]
    h = jnp.dot(s, wt_ref[...], preferred_element_type=jnp.float32) + b_ref[...]
    o_ref[...] = x_ref[...] + jnp.maximum(h, 0.0)


_RB = 1000


def kernel(x, edge_index, W, b):
    src = edge_index[0].astype(jnp.int32)
    dst = edge_index[1].astype(jnp.int32)
    src3 = src.reshape(NW, NCHUNK, CHUNK)
    dst3 = dst.reshape(NW, NCHUNK, CHUNK)
    dst2 = dst.reshape(NW, EPW)

    hist_call, scatter_call = _sc_calls()
    hp = hist_call(dst2)
    p = hp[:, :N].reshape(NW, N, 1)

    grid = N // _RB
    row_spec = pl.BlockSpec((_RB, D), lambda i: (i, 0))
    col_spec = pl.BlockSpec((_RB, 1), lambda i: (i, 0))
    part_spec = pl.BlockSpec((NW, _RB, 1), lambda i: (0, i, 0))

    xt, dis = pl.pallas_call(
        _norm_body,
        grid=(grid,),
        in_specs=[row_spec, part_spec],
        out_specs=[row_spec, col_spec],
        out_shape=[
            jax.ShapeDtypeStruct((N, D), jnp.float32),
            jax.ShapeDtypeStruct((N, 1), jnp.float32),
        ],
    )(x, p)

    parts = scatter_call(xt, src3, dst3)

    out = pl.pallas_call(
        _out_body,
        grid=(grid,),
        in_specs=[
            row_spec, row_spec, row_spec, row_spec, col_spec,
            pl.BlockSpec((D, D), lambda i: (0, 0)),
            pl.BlockSpec((1, D), lambda i: (0, 0)),
        ],
        out_specs=row_spec,
        out_shape=jax.ShapeDtypeStruct((N, D), jnp.float32),
    )(x, parts[0], parts[1], xt, dis, W.T, b.reshape(1, D))
    return out


# speedup vs baseline: 28.8895x; 1.5246x over previous
import functools

import jax
import jax.numpy as jnp
from jax import lax
from jax.experimental import pallas as pl
from jax.experimental.pallas import tpu as pltpu
from jax.experimental.pallas import tpu_sc as plsc

N = 10000
E = 320000
D = 128
NC = 2
NS = 16
NW = NC * NS
EPW = E // NW
CHUNK = 125
NCHUNK = EPW // CHUNK
HROWS = 80
NPAD = 10240
TSTRIPE = NPAD // NS
LASTROWS = N - 15 * TSTRIPE
ZROWS = 128

def _hist_body(dst_hbm, out_hbm, dvals, hist):
    cid = lax.axis_index("c")
    sid = lax.axis_index("s")
    tid = cid * NS + sid

    zeros16 = jnp.zeros((16,), jnp.float32)

    def zb(i, carry):
        hist[pl.ds(i * 16, 16)] = zeros16
        return carry
    lax.fori_loop(0, NPAD // 16, zb, 0)

    pltpu.sync_copy(dst_hbm.at[tid], dvals)

    ones16 = jnp.ones((16,), jnp.float32)

    def hb(i, carry):
        d = dvals[pl.ds(i * 16, 16)]
        plsc.addupdate_scatter(hist, [d], ones16)
        return carry
    lax.fori_loop(0, EPW // 16, hb, 0)

    pltpu.sync_copy(hist, out_hbm.at[tid])


@functools.cache
def _sc_calls():
    mesh = plsc.VectorSubcoreMesh(
        core_axis_name="c", subcore_axis_name="s", num_cores=NC, num_subcores=NS
    )
    params = pltpu.CompilerParams(needs_layout_passes=False)
    hist_call = pl.kernel(
        _hist_body,
        out_type=jax.ShapeDtypeStruct((NW, NPAD), jnp.float32),
        mesh=mesh,
        compiler_params=params,
        scratch_types=[
            pltpu.VMEM((EPW,), jnp.int32),
            pltpu.VMEM((NPAD,), jnp.float32),
        ],
    )
    scatter_call = pl.kernel(
        _scatter_body,
        out_type=jax.ShapeDtypeStruct((NC, N, D), jnp.float32),
        mesh=mesh,
        compiler_params=params,
        scratch_types=[
            pltpu.VMEM((NCHUNK, CHUNK), jnp.int32),
            pltpu.VMEM((NCHUNK, CHUNK), jnp.int32),
            pltpu.VMEM((CHUNK, D), jnp.float32),
            pltpu.SemaphoreType.DMA,
            pltpu.VMEM_SHARED((NPAD, D), jnp.float32),
        ],
    )
    return hist_call, scatter_call


def _scatter_body(xt_hbm, src_hbm, dst_hbm, out_hbm,
                  src_v, dst_v, rows_v, sem, h_sh):
    cid = lax.axis_index("c")
    sid = lax.axis_index("s")
    tid = cid * NS + sid

    pltpu.sync_copy(src_hbm.at[tid], src_v)
    pltpu.sync_copy(dst_hbm.at[tid], dst_v)

    @pl.when(sid < NS - 1)
    def _():
        pltpu.sync_copy(
            xt_hbm.at[pl.ds(sid * TSTRIPE, TSTRIPE)],
            h_sh.at[pl.ds(sid * TSTRIPE, TSTRIPE)],
        )

    @pl.when(sid == NS - 1)
    def _():
        pltpu.sync_copy(
            xt_hbm.at[pl.ds((NS - 1) * TSTRIPE, LASTROWS)],
            h_sh.at[pl.ds((NS - 1) * TSTRIPE, LASTROWS)],
        )

    plsc.subcore_barrier()

    def cb(c, carry):
        pltpu.async_copy(xt_hbm.at[src_v.at[c]], rows_v, sem).wait()
        pltpu.sync_copy(rows_v, h_sh.at[dst_v.at[c]], add=True)
        return carry
    lax.fori_loop(0, NCHUNK, cb, 0)

    plsc.subcore_barrier()

    @pl.when(sid < NS - 1)
    def _():
        pltpu.sync_copy(
            h_sh.at[pl.ds(sid * TSTRIPE, TSTRIPE)],
            out_hbm.at[cid, pl.ds(sid * TSTRIPE, TSTRIPE)],
        )

    @pl.when(sid == NS - 1)
    def _():
        pltpu.sync_copy(
            h_sh.at[pl.ds((NS - 1) * TSTRIPE, LASTROWS)],
            out_hbm.at[cid, pl.ds((NS - 1) * TSTRIPE, LASTROWS)],
        )


def _row_scale(p_ref):
    deg = jnp.sum(p_ref[...], axis=0) + 1.0
    return lax.rsqrt(deg)[:, None]


def _norm_body(x_ref, p_ref, xt_ref):
    xt_ref[...] = x_ref[...] * _row_scale(p_ref)


def _out_body(x_ref, parts_ref, p_ref, w_ref, b_ref, o_ref):
    dis = _row_scale(p_ref)
    x = x_ref[...]
    s = (parts_ref[0] + parts_ref[1] - x * dis) * dis
    h = lax.dot_general(
        s, w_ref[...], (((1,), (1,)), ((), ())),
        preferred_element_type=jnp.float32,
    ) + b_ref[...]
    o_ref[...] = x + jnp.maximum(h, 0.0)


_RB = 1024


def kernel(x, edge_index, W, b):
    src = edge_index[0].astype(jnp.int32)
    dst = edge_index[1].astype(jnp.int32)
    src3 = src.reshape(NW, NCHUNK, CHUNK)
    dst3 = dst.reshape(NW, NCHUNK, CHUNK)
    dst2 = dst.reshape(NW, EPW)

    hist_call, scatter_call = _sc_calls()
    hp = hist_call(dst2)
    ps = hp[:, :N]

    grid = (N + _RB - 1) // _RB
    row_spec = pl.BlockSpec((_RB, D), lambda i: (i, 0))
    hist_spec = pl.BlockSpec((NW, _RB), lambda i: (0, i))

    xt = pl.pallas_call(
        _norm_body,
        grid=(grid,),
        in_specs=[row_spec, hist_spec],
        out_specs=row_spec,
        out_shape=jax.ShapeDtypeStruct((N, D), jnp.float32),
    )(x, ps)

    parts = scatter_call(xt, src3, dst3)

    out = pl.pallas_call(
        _out_body,
        grid=(grid,),
        in_specs=[
            row_spec,
            pl.BlockSpec((NC, _RB, D), lambda i: (0, i, 0)),
            hist_spec,
            pl.BlockSpec((D, D), lambda i: (0, 0)),
            pl.BlockSpec((1, D), lambda i: (0, 0)),
        ],
        out_specs=row_spec,
        out_shape=jax.ShapeDtypeStruct((N, D), jnp.float32),
    )(x, parts, ps, W, b.reshape(1, D))
    return out


# speedup vs baseline: 33.2379x; 1.1505x over previous
import functools

import jax
import jax.numpy as jnp
from jax import lax
from jax.experimental import pallas as pl
from jax.experimental.pallas import tpu as pltpu
from jax.experimental.pallas import tpu_sc as plsc

N = 10000
E = 320000
D = 128
NC = 2
NS = 16
NW = NC * NS
EPW = E // NW
CHUNK = 100
NCHUNK = EPW // CHUNK
NHALF = NCHUNK // 2
HROWS = 80
NPAD = 10240
TSTRIPE = NPAD // NS
LASTROWS = N - 15 * TSTRIPE
ZROWS = 128

def _hist_body(dst_hbm, out_hbm, dvals, hist):
    cid = lax.axis_index("c")
    sid = lax.axis_index("s")
    tid = cid * NS + sid

    zeros16 = jnp.zeros((16,), jnp.float32)

    def zb(i, carry):
        hist[pl.ds(i * 16, 16)] = zeros16
        return carry
    lax.fori_loop(0, NPAD // 16, zb, 0)

    pltpu.sync_copy(dst_hbm.at[tid], dvals)

    ones16 = jnp.ones((16,), jnp.float32)

    def hb(i, carry):
        d = dvals[pl.ds(i * 16, 16)]
        plsc.addupdate_scatter(hist, [d], ones16)
        return carry
    lax.fori_loop(0, EPW // 16, hb, 0)

    pltpu.sync_copy(hist, out_hbm.at[tid])


@functools.cache
def _sc_calls():
    mesh = plsc.VectorSubcoreMesh(
        core_axis_name="c", subcore_axis_name="s", num_cores=NC, num_subcores=NS
    )
    params = pltpu.CompilerParams(needs_layout_passes=False)
    hist_call = pl.kernel(
        _hist_body,
        out_type=jax.ShapeDtypeStruct((NW, NPAD), jnp.float32),
        mesh=mesh,
        compiler_params=params,
        scratch_types=[
            pltpu.VMEM((EPW,), jnp.int32),
            pltpu.VMEM((NPAD,), jnp.float32),
        ],
    )
    scatter_call = pl.kernel(
        _scatter_body,
        out_type=jax.ShapeDtypeStruct((NC, N, D), jnp.float32),
        mesh=mesh,
        compiler_params=params,
        scratch_types=[
            pltpu.VMEM((NHALF, CHUNK), jnp.int32),
            pltpu.VMEM((NHALF, CHUNK), jnp.int32),
            pltpu.VMEM((CHUNK, D), jnp.float32),
            pltpu.VMEM((CHUNK, D), jnp.float32),
            pltpu.SemaphoreType.DMA,
            pltpu.SemaphoreType.DMA,
            pltpu.SemaphoreType.DMA,
            pltpu.SemaphoreType.DMA,
            pltpu.VMEM_SHARED((NPAD, D), jnp.float32),
        ],
    )
    return hist_call, scatter_call


def _scatter_body(xt_hbm, src_hbm, dst_hbm, out_hbm,
                  src_v, dst_v, rows0, rows1, gsem0, gsem1, ssem0, ssem1, h_sh):
    cid = lax.axis_index("c")
    sid = lax.axis_index("s")
    tid = cid * NS + sid

    @pl.when(sid < NS - 1)
    def _():
        pltpu.sync_copy(
            xt_hbm.at[pl.ds(sid * TSTRIPE, TSTRIPE)],
            h_sh.at[pl.ds(sid * TSTRIPE, TSTRIPE)],
        )

    @pl.when(sid == NS - 1)
    def _():
        pltpu.sync_copy(
            xt_hbm.at[pl.ds((NS - 1) * TSTRIPE, LASTROWS)],
            h_sh.at[pl.ds((NS - 1) * TSTRIPE, LASTROWS)],
        )

    plsc.subcore_barrier()

    def _gather(c, rows, sem):
        return pltpu.async_copy(xt_hbm.at[src_v.at[c]], rows, sem)

    def _gather_wait(c, rows, sem):
        pltpu.make_async_copy(xt_hbm.at[src_v.at[c]], rows, sem).wait()

    def _scatter(c, rows, sem):
        return pltpu.async_copy(rows, h_sh.at[dst_v.at[c]], sem, add=True)

    def _scatter_wait(c, rows, sem):
        pltpu.make_async_copy(rows, h_sh.at[dst_v.at[c]], sem).wait()

    def cb(i, carry):
        c = 2 * i
        _gather_wait(c, rows0, gsem0)
        _scatter(c, rows0, ssem0)

        @pl.when(c > 0)
        def _():
            _scatter_wait(c - 1, rows1, ssem1)

        _gather(c + 1, rows1, gsem1)
        _gather_wait(c + 1, rows1, gsem1)
        _scatter(c + 1, rows1, ssem1)
        _scatter_wait(c, rows0, ssem0)

        @pl.when(c + 2 < NHALF)
        def _():
            _gather(c + 2, rows0, gsem0)
        return carry

    for h in range(2):
        pltpu.sync_copy(src_hbm.at[tid, h], src_v)
        pltpu.sync_copy(dst_hbm.at[tid, h], dst_v)
        _gather(0, rows0, gsem0)
        lax.fori_loop(0, NHALF // 2, cb, 0)
        _scatter_wait(NHALF - 1, rows1, ssem1)

    plsc.subcore_barrier()

    @pl.when(sid < NS - 1)
    def _():
        pltpu.sync_copy(
            h_sh.at[pl.ds(sid * TSTRIPE, TSTRIPE)],
            out_hbm.at[cid, pl.ds(sid * TSTRIPE, TSTRIPE)],
        )

    @pl.when(sid == NS - 1)
    def _():
        pltpu.sync_copy(
            h_sh.at[pl.ds((NS - 1) * TSTRIPE, LASTROWS)],
            out_hbm.at[cid, pl.ds((NS - 1) * TSTRIPE, LASTROWS)],
        )


def _row_scale(p_ref):
    deg = jnp.sum(p_ref[...], axis=0) + 1.0
    return lax.rsqrt(deg)[:, None]


def _norm_body(x_ref, p_ref, xt_ref):
    xt_ref[...] = x_ref[...] * _row_scale(p_ref)


def _out_body(x_ref, parts_ref, p_ref, w_ref, b_ref, o_ref):
    dis = _row_scale(p_ref)
    x = x_ref[...]
    s = (parts_ref[0] + parts_ref[1] - x * dis) * dis
    h = lax.dot_general(
        s, w_ref[...], (((1,), (1,)), ((), ())),
        preferred_element_type=jnp.float32,
    ) + b_ref[...]
    o_ref[...] = x + jnp.maximum(h, 0.0)


_RB = 1024


def kernel(x, edge_index, W, b):
    src = edge_index[0].astype(jnp.int32)
    dst = edge_index[1].astype(jnp.int32)
    src3 = src.reshape(NW, 2, NHALF, CHUNK)
    dst3 = dst.reshape(NW, 2, NHALF, CHUNK)
    dst2 = dst.reshape(NW, EPW)

    hist_call, scatter_call = _sc_calls()
    hp = hist_call(dst2)
    ps = hp[:, :N]

    grid = (N + _RB - 1) // _RB
    row_spec = pl.BlockSpec((_RB, D), lambda i: (i, 0))
    hist_spec = pl.BlockSpec((NW, _RB), lambda i: (0, i))

    xt = pl.pallas_call(
        _norm_body,
        grid=(grid,),
        in_specs=[row_spec, hist_spec],
        out_specs=row_spec,
        out_shape=jax.ShapeDtypeStruct((N, D), jnp.float32),
    )(x, ps)

    parts = scatter_call(xt, src3, dst3)

    out = pl.pallas_call(
        _out_body,
        grid=(grid,),
        in_specs=[
            row_spec,
            pl.BlockSpec((NC, _RB, D), lambda i: (0, i, 0)),
            hist_spec,
            pl.BlockSpec((D, D), lambda i: (0, 0)),
            pl.BlockSpec((1, D), lambda i: (0, 0)),
        ],
        out_specs=row_spec,
        out_shape=jax.ShapeDtypeStruct((N, D), jnp.float32),
    )(x, parts, ps, W, b.reshape(1, D))
    return out


# speedup vs baseline: 34.4897x; 1.0377x over previous
import functools

import jax
import jax.numpy as jnp
from jax import lax
from jax.experimental import pallas as pl
from jax.experimental.pallas import tpu as pltpu
from jax.experimental.pallas import tpu_sc as plsc

N = 10000
E = 320000
D = 128
NC = 2
NS = 16
NW = NC * NS
EPW = E // NW
CHUNK = 125
NCHUNK = EPW // CHUNK
NPHASE = 4
NHALF = NCHUNK // NPHASE
HROWS = 80
NPAD = 10240
TSTRIPE = NPAD // NS
LASTROWS = N - 15 * TSTRIPE
ZROWS = 128

def _hist_body(dst_hbm, out_hbm, dvals, hist):
    cid = lax.axis_index("c")
    sid = lax.axis_index("s")
    tid = cid * NS + sid

    zeros16 = jnp.zeros((16,), jnp.float32)

    def zb(i, carry):
        hist[pl.ds(i * 16, 16)] = zeros16
        return carry
    lax.fori_loop(0, NPAD // 16, zb, 0)

    pltpu.sync_copy(dst_hbm.at[tid], dvals)

    ones16 = jnp.ones((16,), jnp.float32)

    def hb(i, carry):
        d = dvals[pl.ds(i * 16, 16)]
        plsc.addupdate_scatter(hist, [d], ones16)
        return carry
    lax.fori_loop(0, EPW // 16, hb, 0)

    pltpu.sync_copy(hist, out_hbm.at[tid])


@functools.cache
def _sc_calls():
    mesh = plsc.VectorSubcoreMesh(
        core_axis_name="c", subcore_axis_name="s", num_cores=NC, num_subcores=NS
    )
    params = pltpu.CompilerParams(needs_layout_passes=False)
    hist_call = pl.kernel(
        _hist_body,
        out_type=jax.ShapeDtypeStruct((NW, NPAD), jnp.float32),
        mesh=mesh,
        compiler_params=params,
        scratch_types=[
            pltpu.VMEM((EPW,), jnp.int32),
            pltpu.VMEM((NPAD,), jnp.float32),
        ],
    )
    scatter_call = pl.kernel(
        _scatter_body,
        out_type=jax.ShapeDtypeStruct((NC, N, D), jnp.float32),
        mesh=mesh,
        compiler_params=params,
        scratch_types=[
            pltpu.VMEM((NHALF, CHUNK), jnp.int32),
            pltpu.VMEM((NHALF, CHUNK), jnp.int32),
            pltpu.VMEM((CHUNK, D), jnp.float32),
            pltpu.VMEM((CHUNK, D), jnp.float32),
            pltpu.SemaphoreType.DMA,
            pltpu.SemaphoreType.DMA,
            pltpu.SemaphoreType.DMA,
            pltpu.SemaphoreType.DMA,
            pltpu.VMEM_SHARED((NPAD, D), jnp.float32),
        ],
    )
    return hist_call, scatter_call


def _scatter_body(xt_hbm, src_hbm, dst_hbm, out_hbm,
                  src_v, dst_v, rows0, rows1, gsem0, gsem1, ssem0, ssem1, h_sh):
    cid = lax.axis_index("c")
    sid = lax.axis_index("s")
    tid = cid * NS + sid

    @pl.when(sid < NS - 1)
    def _():
        pltpu.sync_copy(
            xt_hbm.at[pl.ds(sid * TSTRIPE, TSTRIPE)],
            h_sh.at[pl.ds(sid * TSTRIPE, TSTRIPE)],
        )

    @pl.when(sid == NS - 1)
    def _():
        pltpu.sync_copy(
            xt_hbm.at[pl.ds((NS - 1) * TSTRIPE, LASTROWS)],
            h_sh.at[pl.ds((NS - 1) * TSTRIPE, LASTROWS)],
        )

    plsc.subcore_barrier()

    def _gather(c, rows, sem):
        return pltpu.async_copy(xt_hbm.at[src_v.at[c]], rows, sem)

    def _gather_wait(c, rows, sem):
        pltpu.make_async_copy(xt_hbm.at[src_v.at[c]], rows, sem).wait()

    def _scatter(c, rows, sem):
        return pltpu.async_copy(rows, h_sh.at[dst_v.at[c]], sem, add=True)

    def _scatter_wait(c, rows, sem):
        pltpu.make_async_copy(rows, h_sh.at[dst_v.at[c]], sem).wait()

    def cb(i, carry):
        c = 2 * i
        _gather_wait(c, rows0, gsem0)
        _scatter(c, rows0, ssem0)

        @pl.when(c > 0)
        def _():
            _scatter_wait(c - 1, rows1, ssem1)

        _gather(c + 1, rows1, gsem1)
        _gather_wait(c + 1, rows1, gsem1)
        _scatter(c + 1, rows1, ssem1)
        _scatter_wait(c, rows0, ssem0)

        @pl.when(c + 2 < NHALF)
        def _():
            _gather(c + 2, rows0, gsem0)
        return carry

    for h in range(NPHASE):
        pltpu.sync_copy(src_hbm.at[tid, h], src_v)
        pltpu.sync_copy(dst_hbm.at[tid, h], dst_v)
        _gather(0, rows0, gsem0)
        lax.fori_loop(0, NHALF // 2, cb, 0)
        _scatter_wait(NHALF - 1, rows1, ssem1)

    plsc.subcore_barrier()

    @pl.when(sid < NS - 1)
    def _():
        pltpu.sync_copy(
            h_sh.at[pl.ds(sid * TSTRIPE, TSTRIPE)],
            out_hbm.at[cid, pl.ds(sid * TSTRIPE, TSTRIPE)],
        )

    @pl.when(sid == NS - 1)
    def _():
        pltpu.sync_copy(
            h_sh.at[pl.ds((NS - 1) * TSTRIPE, LASTROWS)],
            out_hbm.at[cid, pl.ds((NS - 1) * TSTRIPE, LASTROWS)],
        )


def _row_scale(p_ref):
    deg = jnp.sum(p_ref[...], axis=0) + 1.0
    return lax.rsqrt(deg)[:, None]


def _norm_body(x_ref, p_ref, xt_ref):
    xt_ref[...] = x_ref[...] * _row_scale(p_ref)


def _out_body(x_ref, parts_ref, p_ref, w_ref, b_ref, o_ref):
    dis = _row_scale(p_ref)
    x = x_ref[...]
    s = (parts_ref[0] + parts_ref[1] - x * dis) * dis
    h = lax.dot_general(
        s, w_ref[...], (((1,), (1,)), ((), ())),
        preferred_element_type=jnp.float32,
    ) + b_ref[...]
    o_ref[...] = x + jnp.maximum(h, 0.0)


_RB = 1024


def kernel(x, edge_index, W, b):
    src = edge_index[0].astype(jnp.int32)
    dst = edge_index[1].astype(jnp.int32)
    src3 = src.reshape(NW, NPHASE, NHALF, CHUNK)
    dst3 = dst.reshape(NW, NPHASE, NHALF, CHUNK)
    dst2 = dst.reshape(NW, EPW)

    hist_call, scatter_call = _sc_calls()
    hp = hist_call(dst2)
    ps = hp[:, :N]

    grid = (N + _RB - 1) // _RB
    row_spec = pl.BlockSpec((_RB, D), lambda i: (i, 0))
    hist_spec = pl.BlockSpec((NW, _RB), lambda i: (0, i))

    xt = pl.pallas_call(
        _norm_body,
        grid=(grid,),
        in_specs=[row_spec, hist_spec],
        out_specs=row_spec,
        out_shape=jax.ShapeDtypeStruct((N, D), jnp.float32),
    )(x, ps)

    parts = scatter_call(xt, src3, dst3)

    out = pl.pallas_call(
        _out_body,
        grid=(grid,),
        in_specs=[
            row_spec,
            pl.BlockSpec((NC, _RB, D), lambda i: (0, i, 0)),
            hist_spec,
            pl.BlockSpec((D, D), lambda i: (0, 0)),
            pl.BlockSpec((1, D), lambda i: (0, 0)),
        ],
        out_specs=row_spec,
        out_shape=jax.ShapeDtypeStruct((N, D), jnp.float32),
    )(x, parts, ps, W, b.reshape(1, D))
    return out


# speedup vs baseline: 34.8362x; 1.0100x over previous
import functools

import jax
import jax.numpy as jnp
from jax import lax
from jax.experimental import pallas as pl
from jax.experimental.pallas import tpu as pltpu
from jax.experimental.pallas import tpu_sc as plsc

N = 10000
E = 320000
D = 128
NC = 2
NS = 16
NW = NC * NS
EPW = E // NW
CHUNK = 125
NCHUNK = EPW // CHUNK
NPHASE = 4
NHALF = NCHUNK // NPHASE
HROWS = 80
NPAD = 10240
TSTRIPE = NPAD // NS
LASTROWS = N - 15 * TSTRIPE
ZROWS = 128

def _hist_body(dst_hbm, out_hbm, dvals, hist):
    cid = lax.axis_index("c")
    sid = lax.axis_index("s")
    tid = cid * NS + sid

    zeros16 = jnp.zeros((16,), jnp.float32)

    def zb(i, carry):
        hist[pl.ds(i * 16, 16)] = zeros16
        return carry
    lax.fori_loop(0, NPAD // 16, zb, 0)

    pltpu.sync_copy(dst_hbm.at[tid], dvals)

    ones16 = jnp.ones((16,), jnp.float32)

    def hb(i, carry):
        d = dvals[pl.ds(i * 16, 16)]
        plsc.addupdate_scatter(hist, [d], ones16)
        return carry
    lax.fori_loop(0, EPW // 16, hb, 0)

    pltpu.sync_copy(hist, out_hbm.at[tid])


@functools.cache
def _sc_calls():
    mesh = plsc.VectorSubcoreMesh(
        core_axis_name="c", subcore_axis_name="s", num_cores=NC, num_subcores=NS
    )
    params = pltpu.CompilerParams(needs_layout_passes=False)
    hist_call = pl.kernel(
        _hist_body,
        out_type=jax.ShapeDtypeStruct((NW, NPAD), jnp.float32),
        mesh=mesh,
        compiler_params=params,
        scratch_types=[
            pltpu.VMEM((EPW,), jnp.int32),
            pltpu.VMEM((NPAD,), jnp.float32),
        ],
    )
    scatter_call = pl.kernel(
        _scatter_body,
        out_type=jax.ShapeDtypeStruct((NC, N, D), jnp.float32),
        mesh=mesh,
        compiler_params=params,
        scratch_types=[
            pltpu.VMEM((NHALF, CHUNK), jnp.int32),
            pltpu.VMEM((NHALF, CHUNK), jnp.int32),
            pltpu.VMEM((CHUNK, D), jnp.float32),
            pltpu.VMEM((CHUNK, D), jnp.float32),
            pltpu.SemaphoreType.DMA,
            pltpu.SemaphoreType.DMA,
            pltpu.SemaphoreType.DMA,
            pltpu.SemaphoreType.DMA,
            pltpu.VMEM_SHARED((NPAD, D), jnp.float32),
        ],
    )
    return hist_call, scatter_call


def _scatter_body(xt_hbm, src_hbm, dst_hbm, out_hbm,
                  src_v, dst_v, rows0, rows1, gsem0, gsem1, ssem0, ssem1, h_sh):
    cid = lax.axis_index("c")
    sid = lax.axis_index("s")
    tid = cid * NS + sid

    @pl.when(sid < NS - 1)
    def _():
        pltpu.sync_copy(
            xt_hbm.at[pl.ds(sid * TSTRIPE, TSTRIPE)],
            h_sh.at[pl.ds(sid * TSTRIPE, TSTRIPE)],
        )

    @pl.when(sid == NS - 1)
    def _():
        pltpu.sync_copy(
            xt_hbm.at[pl.ds((NS - 1) * TSTRIPE, LASTROWS)],
            h_sh.at[pl.ds((NS - 1) * TSTRIPE, LASTROWS)],
        )

    plsc.subcore_barrier()

    def _gather(c, rows, sem):
        return pltpu.async_copy(xt_hbm.at[src_v.at[c]], rows, sem)

    def _gather_wait(c, rows, sem):
        pltpu.make_async_copy(xt_hbm.at[src_v.at[c]], rows, sem).wait()

    def _scatter(c, rows, sem):
        return pltpu.async_copy(rows, h_sh.at[dst_v.at[c]], sem, add=True)

    def _scatter_wait(c, rows, sem):
        pltpu.make_async_copy(rows, h_sh.at[dst_v.at[c]], sem).wait()

    def cb(i, carry):
        c = 2 * i
        _gather_wait(c, rows0, gsem0)
        _scatter(c, rows0, ssem0)

        @pl.when(c > 0)
        def _():
            _scatter_wait(c - 1, rows1, ssem1)

        _gather(c + 1, rows1, gsem1)
        _gather_wait(c + 1, rows1, gsem1)
        _scatter(c + 1, rows1, ssem1)
        _scatter_wait(c, rows0, ssem0)

        @pl.when(c + 2 < NHALF)
        def _():
            _gather(c + 2, rows0, gsem0)
        return carry

    for h in range(NPHASE):
        pltpu.sync_copy(src_hbm.at[tid, h], src_v)
        pltpu.sync_copy(dst_hbm.at[tid, h], dst_v)
        _gather(0, rows0, gsem0)
        lax.fori_loop(0, NHALF // 2, cb, 0)
        _scatter_wait(NHALF - 1, rows1, ssem1)

    plsc.subcore_barrier()

    @pl.when(sid < NS - 1)
    def _():
        pltpu.sync_copy(
            h_sh.at[pl.ds(sid * TSTRIPE, TSTRIPE)],
            out_hbm.at[cid, pl.ds(sid * TSTRIPE, TSTRIPE)],
        )

    @pl.when(sid == NS - 1)
    def _():
        pltpu.sync_copy(
            h_sh.at[pl.ds((NS - 1) * TSTRIPE, LASTROWS)],
            out_hbm.at[cid, pl.ds((NS - 1) * TSTRIPE, LASTROWS)],
        )


def _row_scale(p_ref):
    deg = jnp.sum(p_ref[...], axis=0) + 1.0
    return lax.rsqrt(deg)[:, None]


def _norm_body(x_ref, p_ref, xt_ref):
    xt_ref[...] = x_ref[...] * _row_scale(p_ref)


def _out_body(x_ref, parts_ref, p_ref, w_ref, b_ref, o_ref):
    dis = _row_scale(p_ref)
    x = x_ref[...]
    s = (parts_ref[0] + parts_ref[1] - x * dis) * dis
    h = lax.dot_general(
        s, w_ref[...], (((1,), (1,)), ((), ())),
        preferred_element_type=jnp.float32,
    ) + b_ref[...]
    o_ref[...] = x + jnp.maximum(h, 0.0)


_RB = 1024


def kernel(x, edge_index, W, b):
    src = edge_index[0].astype(jnp.int32)
    dst = edge_index[1].astype(jnp.int32)
    src3 = src.reshape(NW, NPHASE, NHALF, CHUNK)
    dst3 = dst.reshape(NW, NPHASE, NHALF, CHUNK)
    dst2 = dst.reshape(NW, EPW)

    hist_call, scatter_call = _sc_calls()
    ps = hist_call(dst2)

    grid = (N + _RB - 1) // _RB
    row_spec = pl.BlockSpec((_RB, D), lambda i: (i, 0))
    hist_spec = pl.BlockSpec((NW, _RB), lambda i: (0, i))

    xt = pl.pallas_call(
        _norm_body,
        grid=(grid,),
        in_specs=[row_spec, hist_spec],
        out_specs=row_spec,
        out_shape=jax.ShapeDtypeStruct((N, D), jnp.float32),
    )(x, ps)

    parts = scatter_call(xt, src3, dst3)

    out = pl.pallas_call(
        _out_body,
        grid=(grid,),
        in_specs=[
            row_spec,
            pl.BlockSpec((NC, _RB, D), lambda i: (0, i, 0)),
            hist_spec,
            pl.BlockSpec((D, D), lambda i: (0, 0)),
            pl.BlockSpec((1, D), lambda i: (0, 0)),
        ],
        out_specs=row_spec,
        out_shape=jax.ShapeDtypeStruct((N, D), jnp.float32),
    )(x, parts, ps, W, b.reshape(1, D))
    return out


# speedup vs baseline: 34.9000x; 1.0018x over previous
import functools

import jax
import jax.numpy as jnp
from jax import lax
from jax.experimental import pallas as pl
from jax.experimental.pallas import tpu as pltpu
from jax.experimental.pallas import tpu_sc as plsc

N = 10000
E = 320000
D = 128
NC = 2
NS = 16
NW = NC * NS
EPW = E // NW
CHUNK = 125
NCHUNK = EPW // CHUNK
NPHASE = 4
NHALF = NCHUNK // NPHASE
HROWS = 80
NPAD = 10240
TSTRIPE = NPAD // NS
LASTROWS = N - 15 * TSTRIPE

def _hist_body(dst_hbm, out_hbm, dvals, hist):
    cid = lax.axis_index("c")
    sid = lax.axis_index("s")
    tid = cid * NS + sid

    zeros16 = jnp.zeros((16,), jnp.float32)

    def zb(i, carry):
        hist[pl.ds(i * 16, 16)] = zeros16
        return carry
    lax.fori_loop(0, NPAD // 16, zb, 0)

    pltpu.sync_copy(dst_hbm.at[tid], dvals)

    ones16 = jnp.ones((16,), jnp.float32)

    def hb(i, carry):
        d = dvals[pl.ds(i * 16, 16)]
        plsc.addupdate_scatter(hist, [d], ones16)
        return carry
    lax.fori_loop(0, EPW // 16, hb, 0)

    pltpu.sync_copy(hist, out_hbm.at[tid])


@functools.cache
def _sc_calls():
    mesh = plsc.VectorSubcoreMesh(
        core_axis_name="c", subcore_axis_name="s", num_cores=NC, num_subcores=NS
    )
    params = pltpu.CompilerParams(needs_layout_passes=False)
    hist_call = pl.kernel(
        _hist_body,
        out_type=jax.ShapeDtypeStruct((NW, NPAD), jnp.float32),
        mesh=mesh,
        compiler_params=params,
        scratch_types=[
            pltpu.VMEM((EPW,), jnp.int32),
            pltpu.VMEM((NPAD,), jnp.float32),
        ],
    )
    scatter_call = pl.kernel(
        _scatter_body,
        out_type=jax.ShapeDtypeStruct((NC, N, D), jnp.float32),
        mesh=mesh,
        compiler_params=params,
        scratch_types=[
            pltpu.VMEM((NHALF, CHUNK), jnp.int32),
            pltpu.VMEM((NHALF, CHUNK), jnp.int32),
            pltpu.VMEM((CHUNK, D), jnp.float32),
            pltpu.VMEM((CHUNK, D), jnp.float32),
            pltpu.SemaphoreType.DMA,
            pltpu.SemaphoreType.DMA,
            pltpu.SemaphoreType.DMA,
            pltpu.SemaphoreType.DMA,
            pltpu.VMEM_SHARED((NPAD, D), jnp.float32),
        ],
    )
    return hist_call, scatter_call


def _scatter_body(xt_hbm, src_hbm, dst_hbm, out_hbm,
                  src_v, dst_v, rows0, rows1, gsem0, gsem1, ssem0, ssem1, h_sh):
    cid = lax.axis_index("c")
    sid = lax.axis_index("s")
    tid = cid * NS + sid

    @pl.when(sid < NS - 1)
    def _():
        pltpu.sync_copy(
            xt_hbm.at[pl.ds(sid * TSTRIPE, TSTRIPE)],
            h_sh.at[pl.ds(sid * TSTRIPE, TSTRIPE)],
        )

    @pl.when(sid == NS - 1)
    def _():
        pltpu.sync_copy(
            xt_hbm.at[pl.ds((NS - 1) * TSTRIPE, LASTROWS)],
            h_sh.at[pl.ds((NS - 1) * TSTRIPE, LASTROWS)],
        )

    plsc.subcore_barrier()

    def _gather(c, rows, sem):
        return pltpu.async_copy(xt_hbm.at[src_v.at[c]], rows, sem)

    def _gather_wait(c, rows, sem):
        pltpu.make_async_copy(xt_hbm.at[src_v.at[c]], rows, sem).wait()

    def _scatter(c, rows, sem):
        return pltpu.async_copy(rows, h_sh.at[dst_v.at[c]], sem, add=True)

    def _scatter_wait(c, rows, sem):
        pltpu.make_async_copy(rows, h_sh.at[dst_v.at[c]], sem).wait()

    def cb(i, carry):
        c = 2 * i
        _gather_wait(c, rows0, gsem0)
        _scatter(c, rows0, ssem0)

        @pl.when(c > 0)
        def _():
            _scatter_wait(c - 1, rows1, ssem1)

        _gather(c + 1, rows1, gsem1)
        _gather_wait(c + 1, rows1, gsem1)
        _scatter(c + 1, rows1, ssem1)
        _scatter_wait(c, rows0, ssem0)

        @pl.when(c + 2 < NHALF)
        def _():
            _gather(c + 2, rows0, gsem0)
        return carry

    for h in range(NPHASE):
        pltpu.sync_copy(src_hbm.at[tid, h], src_v)
        pltpu.sync_copy(dst_hbm.at[tid, h], dst_v)
        _gather(0, rows0, gsem0)
        lax.fori_loop(0, NHALF // 2, cb, 0)
        _scatter_wait(NHALF - 1, rows1, ssem1)

    plsc.subcore_barrier()

    @pl.when(sid < NS - 1)
    def _():
        pltpu.sync_copy(
            h_sh.at[pl.ds(sid * TSTRIPE, TSTRIPE)],
            out_hbm.at[cid, pl.ds(sid * TSTRIPE, TSTRIPE)],
        )

    @pl.when(sid == NS - 1)
    def _():
        pltpu.sync_copy(
            h_sh.at[pl.ds((NS - 1) * TSTRIPE, LASTROWS)],
            out_hbm.at[cid, pl.ds((NS - 1) * TSTRIPE, LASTROWS)],
        )


def _row_scale(p_ref):
    deg = jnp.sum(p_ref[...], axis=0) + 1.0
    return lax.rsqrt(deg)[:, None]


def _norm_body(x_ref, p_ref, xt_ref):
    xt_ref[...] = x_ref[...] * _row_scale(p_ref)


def _out_body(x_ref, parts_ref, p_ref, w_ref, b_ref, o_ref):
    dis = _row_scale(p_ref)
    x = x_ref[...]
    s = (parts_ref[0] + parts_ref[1] - x * dis) * dis
    h = lax.dot_general(
        s, w_ref[...], (((1,), (1,)), ((), ())),
        preferred_element_type=jnp.float32,
    ) + b_ref[...]
    o_ref[...] = x + jnp.maximum(h, 0.0)


_RB = 1024


def kernel(x, edge_index, W, b):
    src = edge_index[0].astype(jnp.int32)
    dst = edge_index[1].astype(jnp.int32)
    src3 = src.reshape(NW, NPHASE, NHALF, CHUNK)
    dst3 = dst.reshape(NW, NPHASE, NHALF, CHUNK)
    dst2 = dst.reshape(NW, EPW)

    hist_call, scatter_call = _sc_calls()
    ps = hist_call(dst2)

    grid = (N + _RB - 1) // _RB
    row_spec = pl.BlockSpec((_RB, D), lambda i: (i, 0))
    hist_spec = pl.BlockSpec((NW, _RB), lambda i: (0, i))

    xt = pl.pallas_call(
        _norm_body,
        grid=(grid,),
        in_specs=[row_spec, hist_spec],
        out_specs=row_spec,
        out_shape=jax.ShapeDtypeStruct((N, D), jnp.float32),
    )(x, ps)

    parts = scatter_call(xt, src3, dst3)

    out = pl.pallas_call(
        _out_body,
        grid=(grid,),
        in_specs=[
            row_spec,
            pl.BlockSpec((NC, _RB, D), lambda i: (0, i, 0)),
            hist_spec,
            pl.BlockSpec((D, D), lambda i: (0, 0)),
            pl.BlockSpec((1, D), lambda i: (0, 0)),
        ],
        out_specs=row_spec,
        out_shape=jax.ShapeDtypeStruct((N, D), jnp.float32),
    )(x, parts, ps, W, b.reshape(1, D))
    return out
